# baseline (speedup 1.0000x reference)
import jax
import jax.numpy as jnp
from jax.experimental import pallas as pl

_VARIANCES = jnp.array([0.1, 0.1, 0.2, 0.2], dtype=jnp.float32)
_C = 4
_M = 100
_SCORE_T = 0.05
_IOU_T = 0.5
_NEG = -1e9


def _decode(anchors, deltas):
    aw = anchors[..., 3] - anchors[..., 1]
    ah = anchors[..., 2] - anchors[..., 0]
    acx = anchors[..., 1] + 0.5 * aw
    acy = anchors[..., 0] + 0.5 * ah
    bw = jnp.exp(deltas[..., 3]) * aw
    bh = jnp.exp(deltas[..., 2]) * ah
    bcx = deltas[..., 1] * aw + acx
    bcy = deltas[..., 0] * ah + acy
    y1 = bcy - 0.5 * bh
    x1 = bcx - 0.5 * bw
    y2 = bh + y1
    x2 = bw + x1
    return jnp.stack([y1, x1, y2, x2], axis=-1)


def _iou_one(box, boxes):
    yy1 = jnp.maximum(box[0], boxes[:, 0])
    xx1 = jnp.maximum(box[1], boxes[:, 1])
    yy2 = jnp.minimum(box[2], boxes[:, 2])
    xx2 = jnp.minimum(box[3], boxes[:, 3])
    inter = jnp.maximum(yy2 - yy1, 0.0) * jnp.maximum(xx2 - xx1, 0.0)
    a1 = jnp.maximum(box[2] - box[0], 0.0) * jnp.maximum(box[3] - box[1], 0.0)
    a2 = jnp.maximum(boxes[:, 2] - boxes[:, 0], 0.0) * jnp.maximum(boxes[:, 3] - boxes[:, 1], 0.0)
    union = a1 + a2 - inter
    return jnp.where(union > 0.0, inter / union, 0.0)


def _nms_one(boxes, scores):
    scores = jnp.where(scores > _SCORE_T, scores, _NEG)

    def body(sc, _):
        idx = jnp.argmax(sc)
        best = sc[idx]
        box = boxes[idx]
        ious = _iou_one(box, boxes)
        sup = ious > _IOU_T
        new_sc = jnp.where(sup, _NEG, sc)
        new_sc = new_sc.at[idx].set(_NEG)
        valid = best > _NEG / 2.0
        return new_sc, (box, jnp.where(valid, best, _NEG), valid)

    _, (sb, ss, vv) = jax.lax.scan(body, scores, None, length=_M)
    return sb, ss, vv


def _combined_one(boxes_b, scores_b):
    sb, ss, _ = jax.vmap(_nms_one, in_axes=(1, 1))(boxes_b, scores_b)
    classes = jnp.broadcast_to(jnp.arange(_C, dtype=jnp.float32)[:, None], ss.shape)
    fs = ss.reshape(-1)
    fb = sb.reshape(-1, 4)
    fc = classes.reshape(-1)
    ts, ti = jax.lax.top_k(fs, _M)
    valid = ts > _NEG / 2.0
    return (jnp.where(valid[:, None], fb[ti], 0.0), jnp.where(valid, ts, 0.0),
            jnp.where(valid, fc[ti], 0.0))


def kernel(roi_bboxes, pred_deltas, pred_label_probs):
    def copy_k(x_ref, o_ref):
        o_ref[...] = x_ref[...]

    pf = pred_label_probs.reshape(2500, 128)
    probs = pl.pallas_call(
        copy_k,
        out_shape=jax.ShapeDtypeStruct(pf.shape, pf.dtype),
    )(pf).reshape(pred_label_probs.shape)

    B = roi_bboxes.shape[0]
    pd = pred_deltas.reshape(B, -1, _C, 4) * _VARIANCES
    anc = jnp.tile(roi_bboxes[:, :, None, :], (1, 1, _C, 1))
    boxes = jnp.clip(_decode(anc, pd), 0.0, 1.0)
    lab_map = jnp.argmax(probs, axis=-1)[..., None]
    labels = jnp.where(lab_map != 0, probs, jnp.zeros_like(probs))
    fb, fs, fl = jax.vmap(_combined_one)(boxes, labels)
    return (jax.lax.stop_gradient(fb), jax.lax.stop_gradient(fl),
            jax.lax.stop_gradient(fs))


# speedup vs baseline: 95.7717x; 95.7717x over previous
import functools

import jax
import jax.numpy as jnp
from jax import lax
from jax.experimental import pallas as pl
from jax.experimental.pallas import tpu as pltpu
from jax.experimental.pallas import tpu_sc as plsc

NEG = -1e9
B = 4
C = 4
N = 20000
NP = 20480
NV = NP // 16
L2N = NV // 16
M = 100
KS = 128
SCORE_T = 0.05
IOU_T = 0.5
F32 = jnp.float32
I32 = jnp.int32


def _tc_decode_body(roi_ref, d_ref, p_ref, sc_ref, bx_ref):
    c = pl.program_id(0) % C
    ay1 = roi_ref[0, 0:1, :]
    ax1 = roi_ref[0, 1:2, :]
    ay2 = roi_ref[0, 2:3, :]
    ax2 = roi_ref[0, 3:4, :]
    dy = d_ref[0, 0, 0:1, :] * 0.1
    dx = d_ref[0, 0, 1:2, :] * 0.1
    dh = d_ref[0, 0, 2:3, :] * 0.2
    dw = d_ref[0, 0, 3:4, :] * 0.2
    aw = ax2 - ax1
    ah = ay2 - ay1
    acx = ax1 + 0.5 * aw
    acy = ay1 + 0.5 * ah
    bw = jnp.exp(dw) * aw
    bh = jnp.exp(dh) * ah
    bcx = dx * aw + acx
    bcy = dy * ah + acy
    y1 = bcy - 0.5 * bh
    x1 = bcx - 0.5 * bw
    y2 = bh + y1
    x2 = bw + x1
    bx_ref[0, 0:1, :] = jnp.clip(y1, 0.0, 1.0)
    bx_ref[0, 1:2, :] = jnp.clip(x1, 0.0, 1.0)
    bx_ref[0, 2:3, :] = jnp.clip(y2, 0.0, 1.0)
    bx_ref[0, 3:4, :] = jnp.clip(x2, 0.0, 1.0)

    p0 = p_ref[0, 0:1, :]
    p1 = p_ref[0, 1:2, :]
    p2 = p_ref[0, 2:3, :]
    p3 = p_ref[0, 3:4, :]
    pc = jnp.where(c == 0, p0, jnp.where(c == 1, p1, jnp.where(c == 2, p2, p3)))
    bg = p0 >= jnp.maximum(p1, jnp.maximum(p2, p3))
    s = jnp.where(bg, 0.0, pc)
    sc_ref[0, 0:1, :] = jnp.where(s > SCORE_T, s, NEG)


def _tc_decode(roi_t, d_t, p_t):
    return pl.pallas_call(
        _tc_decode_body,
        grid=(B * C,),
        in_specs=[
            pl.BlockSpec((1, 4, NP), lambda g: (g // C, 0, 0)),
            pl.BlockSpec((1, 1, 4, NP), lambda g: (g // C, g % C, 0, 0)),
            pl.BlockSpec((1, 4, NP), lambda g: (g // C, 0, 0)),
        ],
        out_specs=[
            pl.BlockSpec((1, 1, NP), lambda g: (g, 0, 0)),
            pl.BlockSpec((1, 4, NP), lambda g: (g, 0, 0)),
        ],
        out_shape=[
            jax.ShapeDtypeStruct((B * C, 1, NP), F32),
            jax.ShapeDtypeStruct((B * C, 4, NP), F32),
        ],
    )(roi_t, d_t, p_t)


def _wid():
    return lax.axis_index("s") * 2 + lax.axis_index("c")


def _smax(v):
    return lax.reduce_max(v, (0,))


def _arglane(v, top):
    ii = lax.iota(I32, 16)
    return lax.reduce_min(jnp.where(v == top, ii, 16), (0,))


def _bc(x):
    return jnp.broadcast_to(x, (16,))


def _st1(ref, idxs, val):
    mask = lax.iota(I32, 16) < 1
    plsc.store_scatter(ref, [_bc(jnp.asarray(i, I32)) for i in idxs],
                       _bc(val), mask=mask)


def _ld1(ref, idxs):
    return plsc.load_gather(ref, [_bc(jnp.asarray(i, I32)) for i in idxs])


def _mesh():
    return plsc.VectorSubcoreMesh(core_axis_name="c", subcore_axis_name="s")


def _sc_nms_body(sc_hbm, bx_hbm, oks_hbm, okb_hbm,
                 s_v, b_v, l1_v, l2_v, l3_v, ks_v, kb_v):
    wid = _wid()

    @pl.when(wid < B * C)
    def _():
        bb = lax.shift_right_logical(wid, 2)
        cc = jnp.bitwise_and(wid, 3)
        pltpu.sync_copy(sc_hbm.at[wid], s_v)
        pltpu.sync_copy(bx_hbm.at[wid], b_v)

        def init_kept(i, _):
            kb_v[0, pl.ds(i * 16, 16)] = jnp.full((16,), -4.0, F32)
            kb_v[1, pl.ds(i * 16, 16)] = jnp.full((16,), -4.0, F32)
            kb_v[2, pl.ds(i * 16, 16)] = jnp.full((16,), -4.0, F32)
            kb_v[3, pl.ds(i * 16, 16)] = jnp.full((16,), -4.0, F32)
            ks_v[pl.ds(i * 16, 16)] = jnp.full((16,), NEG, F32)
            return 0
        lax.fori_loop(0, KS // 16, init_kept, 0)

        def build1(i, _):
            _st1(l1_v, [i], _smax(s_v[pl.ds(i * 16, 16)]))
            return 0
        lax.fori_loop(0, NV, build1, 0)

        def build2(i, _):
            _st1(l2_v, [i], _smax(l1_v[pl.ds(i * 16, 16)]))
            return 0
        lax.fori_loop(0, L2N, build2, 0)

        l3_v[pl.ds(0, 16)] = jnp.full((16,), NEG, F32)

        def build3(i, _):
            _st1(l3_v, [i], _smax(l2_v[pl.ds(i * 16, 16)]))
            return 0
        lax.fori_loop(0, L2N // 16, build3, 0)

        def cond(carry):
            k, cont = carry
            return jnp.logical_and(k < M, cont > 0)

        def body(carry):
            k, _ = carry
            v3 = l3_v[pl.ds(0, 16)]
            top = _smax(v3)
            valid = top > NEG / 2.0
            l3 = _arglane(v3, top)
            v2 = l2_v[pl.ds(l3 * 16, 16)]
            j1 = l3 * 16 + _arglane(v2, top)
            v1 = l1_v[pl.ds(j1 * 16, 16)]
            blk = j1 * 16 + _arglane(v1, top)
            vs = s_v[pl.ds(blk * 16, 16)]
            n = blk * 16 + _arglane(vs, top)

            cy1 = _ld1(b_v, [0, n])
            cx1 = _ld1(b_v, [1, n])
            cy2 = _ld1(b_v, [2, n])
            cx2 = _ld1(b_v, [3, n])
            a1 = jnp.maximum(cy2 - cy1, 0.0) * jnp.maximum(cx2 - cx1, 0.0)
            acc = jnp.zeros((16,), F32)
            for vv in range(KS // 16):
                ky1 = kb_v[0, pl.ds(vv * 16, 16)]
                kx1 = kb_v[1, pl.ds(vv * 16, 16)]
                ky2 = kb_v[2, pl.ds(vv * 16, 16)]
                kx2 = kb_v[3, pl.ds(vv * 16, 16)]
                yy1 = jnp.maximum(cy1, ky1)
                xx1 = jnp.maximum(cx1, kx1)
                yy2 = jnp.minimum(cy2, ky2)
                xx2 = jnp.minimum(cx2, kx2)
                inter = jnp.maximum(yy2 - yy1, 0.0) * jnp.maximum(xx2 - xx1, 0.0)
                a2 = jnp.maximum(ky2 - ky1, 0.0) * jnp.maximum(kx2 - kx1, 0.0)
                union = a1 + a2 - inter
                iou = jnp.where(union > 0.0, inter / union, 0.0)
                acc = jnp.maximum(acc, iou)
            sup = _smax(acc) > IOU_T
            keep = jnp.logical_and(valid, jnp.logical_not(sup))

            @pl.when(valid)
            def _():
                _st1(s_v, [n], jnp.float32(NEG))
                _st1(l1_v, [blk], _smax(s_v[pl.ds(blk * 16, 16)]))
                _st1(l2_v, [j1], _smax(l1_v[pl.ds(j1 * 16, 16)]))
                _st1(l3_v, [l3], _smax(l2_v[pl.ds(l3 * 16, 16)]))

            @pl.when(keep)
            def _():
                _st1(kb_v, [0, k], cy1)
                _st1(kb_v, [1, k], cx1)
                _st1(kb_v, [2, k], cy2)
                _st1(kb_v, [3, k], cx2)
                _st1(ks_v, [k], top)

            return (k + keep.astype(I32), valid.astype(I32))

        lax.while_loop(cond, body, (jnp.int32(0), jnp.int32(1)))

        pltpu.sync_copy(ks_v, oks_hbm.at[bb, pl.ds(cc * KS, KS)])
        pltpu.sync_copy(kb_v, okb_hbm.at[bb, cc])


def _sc_nms(scores, boxes):
    f = functools.partial(
        pl.kernel,
        mesh=_mesh(),
        compiler_params=pltpu.CompilerParams(needs_layout_passes=False),
        out_type=[
            jax.ShapeDtypeStruct((B, C * KS), F32),
            jax.ShapeDtypeStruct((B, C, 4, KS), F32),
        ],
        scratch_types=[
            pltpu.VMEM((NP,), F32),
            pltpu.VMEM((4, NP), F32),
            pltpu.VMEM((NV,), F32),
            pltpu.VMEM((L2N,), F32),
            pltpu.VMEM((16,), F32),
            pltpu.VMEM((KS,), F32),
            pltpu.VMEM((4, KS), F32),
        ],
    )(_sc_nms_body)
    return f(scores, boxes)


def _sc_merge_body(ks_hbm, kb_hbm, ob_hbm, ol_hbm, os_hbm,
                   s4_v, b4_v, l1_v, ob_v, ol_v, os_v):
    wid = _wid()

    @pl.when(wid < B)
    def _():
        pltpu.sync_copy(ks_hbm.at[wid], s4_v)
        pltpu.sync_copy(kb_hbm.at[wid], b4_v)

        nv = C * KS // 16

        def build1(i, _):
            _st1(l1_v, [i], _smax(s4_v[pl.ds(i * 16, 16)]))
            return 0
        lax.fori_loop(0, nv, build1, 0)

        def zo(i, _):
            ob_v[pl.ds(i * 16, 16)] = jnp.zeros((16,), F32)
            return 0
        lax.fori_loop(0, 4 * M // 16, zo, 0)
        for i in range(6):
            ol_v[pl.ds(i * 16, 16)] = jnp.zeros((16,), F32)
            os_v[pl.ds(i * 16, 16)] = jnp.zeros((16,), F32)
        ol_v[pl.ds(104 - 16, 16)] = jnp.zeros((16,), F32)
        os_v[pl.ds(104 - 16, 16)] = jnp.zeros((16,), F32)

        def emit(i, _):
            va = l1_v[pl.ds(0, 16)]
            vb = l1_v[pl.ds(16, 16)]
            top = jnp.maximum(_smax(va), _smax(vb))
            ii = lax.iota(I32, 16)
            ja = lax.reduce_min(jnp.where(va == top, ii, 99), (0,))
            jb = lax.reduce_min(jnp.where(vb == top, ii + 16, 99), (0,))
            j = jnp.minimum(ja, jb)
            vsc = s4_v[pl.ds(j * 16, 16)]
            n = j * 16 + _arglane(vsc, top)
            valid = top > NEG / 2.0

            cls = lax.shift_right_logical(n, 7)
            slot = jnp.bitwise_and(n, KS - 1)
            y1 = _ld1(b4_v, [cls, 0, slot])
            x1 = _ld1(b4_v, [cls, 1, slot])
            y2 = _ld1(b4_v, [cls, 2, slot])
            x2 = _ld1(b4_v, [cls, 3, slot])

            @pl.when(valid)
            def _():
                _st1(s4_v, [n], jnp.float32(NEG))
                _st1(l1_v, [j], _smax(s4_v[pl.ds(j * 16, 16)]))
                _st1(ob_v, [4 * i], y1)
                _st1(ob_v, [4 * i + 1], x1)
                _st1(ob_v, [4 * i + 2], y2)
                _st1(ob_v, [4 * i + 3], x2)
                _st1(ol_v, [i], cls.astype(F32))
                _st1(os_v, [i], top)
            return 0

        lax.fori_loop(0, M, emit, 0)

        pltpu.sync_copy(ob_v, ob_hbm.at[wid])
        pltpu.sync_copy(ol_v, ol_hbm.at[wid])
        pltpu.sync_copy(os_v, os_hbm.at[wid])


def _sc_merge(kscores, kboxes):
    f = functools.partial(
        pl.kernel,
        mesh=_mesh(),
        compiler_params=pltpu.CompilerParams(needs_layout_passes=False),
        out_type=[
            jax.ShapeDtypeStruct((B, 4 * M), F32),
            jax.ShapeDtypeStruct((B, 104), F32),
            jax.ShapeDtypeStruct((B, 104), F32),
        ],
        scratch_types=[
            pltpu.VMEM((C * KS,), F32),
            pltpu.VMEM((C, 4, KS), F32),
            pltpu.VMEM((C * KS // 16,), F32),
            pltpu.VMEM((4 * M,), F32),
            pltpu.VMEM((104,), F32),
            pltpu.VMEM((104,), F32),
        ],
    )(_sc_merge_body)
    return f(kscores, kboxes)


def kernel(roi_bboxes, pred_deltas, pred_label_probs):
    pad = NP - N
    roi_t = jnp.pad(jnp.transpose(roi_bboxes, (0, 2, 1)), ((0, 0), (0, 0), (0, pad)))
    d = pred_deltas.reshape(B, N, C, 4)
    d_t = jnp.pad(jnp.transpose(d, (0, 2, 3, 1)), ((0, 0), (0, 0), (0, 0), (0, pad)))
    p_t = jnp.pad(jnp.transpose(pred_label_probs, (0, 2, 1)), ((0, 0), (0, 0), (0, pad)))

    scores, boxes = _tc_decode(roi_t, d_t, p_t)
    scores = scores.reshape(B * C, NP)
    ks, kb = _sc_nms(scores, boxes)
    ob, ol, os_ = _sc_merge(ks, kb)

    final_boxes = ob.reshape(B, M, 4)
    final_labels = ol[:, :M]
    final_scores = os_[:, :M]
    return (final_boxes, final_labels, final_scores)


# speedup vs baseline: 108.5641x; 1.1336x over previous
import functools

import jax
import jax.numpy as jnp
from jax import lax
from jax.experimental import pallas as pl
from jax.experimental.pallas import tpu as pltpu
from jax.experimental.pallas import tpu_sc as plsc

NEG = -1e9
B = 4
C = 4
N = 20000
NP = 20480
NV = NP // 16
L2N = NV // 16
M = 100
KS = 128
SCORE_T = 0.05
IOU_T = 0.5
F32 = jnp.float32
I32 = jnp.int32


def _tc_decode_body(roi_ref, d_ref, p_ref, sc_ref, bx_ref):
    ay1 = roi_ref[0, 0:1, :]
    ax1 = roi_ref[0, 1:2, :]
    ay2 = roi_ref[0, 2:3, :]
    ax2 = roi_ref[0, 3:4, :]
    aw = ax2 - ax1
    ah = ay2 - ay1
    acx = ax1 + 0.5 * aw
    acy = ay1 + 0.5 * ah
    p0 = p_ref[0, 0:1, :]
    p1 = p_ref[0, 1:2, :]
    p2 = p_ref[0, 2:3, :]
    p3 = p_ref[0, 3:4, :]
    bg = p0 >= jnp.maximum(p1, jnp.maximum(p2, p3))
    for c in range(C):
        dy = d_ref[0, c, 0:1, :] * 0.1
        dx = d_ref[0, c, 1:2, :] * 0.1
        dh = d_ref[0, c, 2:3, :] * 0.2
        dw = d_ref[0, c, 3:4, :] * 0.2
        bw = jnp.exp(dw) * aw
        bh = jnp.exp(dh) * ah
        bcx = dx * aw + acx
        bcy = dy * ah + acy
        y1 = bcy - 0.5 * bh
        x1 = bcx - 0.5 * bw
        y2 = bh + y1
        x2 = bw + x1
        bx_ref[0, c, 0:1, :] = jnp.clip(y1, 0.0, 1.0)
        bx_ref[0, c, 1:2, :] = jnp.clip(x1, 0.0, 1.0)
        bx_ref[0, c, 2:3, :] = jnp.clip(y2, 0.0, 1.0)
        bx_ref[0, c, 3:4, :] = jnp.clip(x2, 0.0, 1.0)
        pc = (p0, p1, p2, p3)[c]
        s = jnp.where(bg, 0.0, pc)
        sc_ref[0, c:c + 1, :] = jnp.where(s > SCORE_T, s, NEG)


def _tc_decode(roi_t, d_t, p_t):
    return pl.pallas_call(
        _tc_decode_body,
        grid=(B,),
        in_specs=[
            pl.BlockSpec((1, 4, NP), lambda g: (g, 0, 0)),
            pl.BlockSpec((1, C, 4, NP), lambda g: (g, 0, 0, 0)),
            pl.BlockSpec((1, 4, NP), lambda g: (g, 0, 0)),
        ],
        out_specs=[
            pl.BlockSpec((1, C, NP), lambda g: (g, 0, 0)),
            pl.BlockSpec((1, C, 4, NP), lambda g: (g, 0, 0, 0)),
        ],
        out_shape=[
            jax.ShapeDtypeStruct((B, C, NP), F32),
            jax.ShapeDtypeStruct((B, C, 4, NP), F32),
        ],
    )(roi_t, d_t, p_t)


def _wid():
    return lax.axis_index("s") * 2 + lax.axis_index("c")


def _smax(v):
    return lax.reduce_max(v, (0,))


def _arglane(v, top):
    return plsc.all_reduce_ffs(v == top)[0]


def _bc(x):
    return jnp.broadcast_to(x, (16,))


def _st1(ref, idxs, val):
    mask = lax.iota(I32, 16) < 1
    plsc.store_scatter(ref, [_bc(jnp.asarray(i, I32)) for i in idxs],
                       _bc(val), mask=mask)


def _ld1(ref, idxs):
    return plsc.load_gather(ref, [_bc(jnp.asarray(i, I32)) for i in idxs])


def _mesh():
    return plsc.VectorSubcoreMesh(core_axis_name="c", subcore_axis_name="s")


def _sc_nms_body(sc_hbm, bx_hbm, oks_hbm, okb_hbm,
                 s_v, b_v, l1_v, l2_v, l3_v, ks_v, kb_v, sem):
    wid = _wid()

    @pl.when(wid < B * C)
    def _():
        bb = lax.shift_right_logical(wid, 2)
        cc = jnp.bitwise_and(wid, 3)
        box_cp = pltpu.async_copy(bx_hbm.at[wid], b_v, sem)
        pltpu.sync_copy(sc_hbm.at[wid], s_v)

        for i in range(KS // 16):
            kb_v[0, pl.ds(i * 16, 16)] = jnp.full((16,), -4.0, F32)
            kb_v[1, pl.ds(i * 16, 16)] = jnp.full((16,), -4.0, F32)
            kb_v[2, pl.ds(i * 16, 16)] = jnp.full((16,), -4.0, F32)
            kb_v[3, pl.ds(i * 16, 16)] = jnp.full((16,), -4.0, F32)
            ks_v[pl.ds(i * 16, 16)] = jnp.full((16,), NEG, F32)

        def build1(i, _):
            for u in range(8):
                g = i * 8 + u
                _st1(l1_v, [g], _smax(s_v[pl.ds(g * 16, 16)]))
            return 0
        lax.fori_loop(0, NV // 8, build1, 0)

        def build2(i, _):
            for u in range(8):
                g = i * 8 + u
                _st1(l2_v, [g], _smax(l1_v[pl.ds(g * 16, 16)]))
            return 0
        lax.fori_loop(0, L2N // 8, build2, 0)

        l3_v[pl.ds(0, 16)] = jnp.full((16,), NEG, F32)
        for g in range(L2N // 16):
            _st1(l3_v, [g], _smax(l2_v[pl.ds(g * 16, 16)]))

        box_cp.wait()

        def cond(carry):
            k, cont = carry
            return jnp.logical_and(k < M, cont > 0)

        def body(carry):
            k, _ = carry
            ii = lax.iota(I32, 16)
            v3 = l3_v[pl.ds(0, 16)]
            top = _smax(v3)
            valid = top > NEG / 2.0
            l3 = _arglane(v3, top)
            v2 = l2_v[pl.ds(l3 * 16, 16)]
            i2 = _arglane(v2, top)
            j1 = l3 * 16 + i2
            v1 = l1_v[pl.ds(j1 * 16, 16)]
            i1 = _arglane(v1, top)
            blk = j1 * 16 + i1
            vs = s_v[pl.ds(blk * 16, 16)]
            i0 = _arglane(vs, top)
            n = blk * 16 + i0

            cy1 = _ld1(b_v, [0, n])
            cx1 = _ld1(b_v, [1, n])
            cy2 = _ld1(b_v, [2, n])
            cx2 = _ld1(b_v, [3, n])
            a1 = jnp.maximum(cy2 - cy1, 0.0) * jnp.maximum(cx2 - cx1, 0.0)

            def iou_vreg(vv, acc):
                ky1 = kb_v[0, pl.ds(vv * 16, 16)]
                kx1 = kb_v[1, pl.ds(vv * 16, 16)]
                ky2 = kb_v[2, pl.ds(vv * 16, 16)]
                kx2 = kb_v[3, pl.ds(vv * 16, 16)]
                yy1 = jnp.maximum(cy1, ky1)
                xx1 = jnp.maximum(cx1, kx1)
                yy2 = jnp.minimum(cy2, ky2)
                xx2 = jnp.minimum(cx2, kx2)
                inter = jnp.maximum(yy2 - yy1, 0.0) * jnp.maximum(xx2 - xx1, 0.0)
                a2 = jnp.maximum(ky2 - ky1, 0.0) * jnp.maximum(kx2 - kx1, 0.0)
                union = a1 + a2 - inter
                iou = jnp.where(union > 0.0, inter / union, 0.0)
                return jnp.maximum(acc, iou)

            kv = lax.shift_right_logical(k + 15, 4)
            acc = lax.fori_loop(0, kv, iou_vreg, jnp.zeros((16,), F32))
            sup = _smax(acc) > IOU_T
            keep = jnp.logical_and(valid, jnp.logical_not(sup))

            @pl.when(valid)
            def _():
                vs2 = jnp.where(ii == i0, NEG, vs)
                s_v[pl.ds(blk * 16, 16)] = vs2
                m1 = _smax(vs2)
                v1n = jnp.where(ii == i1, m1, v1)
                l1_v[pl.ds(j1 * 16, 16)] = v1n
                m2 = _smax(v1n)
                v2n = jnp.where(ii == i2, m2, v2)
                l2_v[pl.ds(l3 * 16, 16)] = v2n
                m3 = _smax(v2n)
                l3_v[pl.ds(0, 16)] = jnp.where(ii == l3, m3, v3)

            @pl.when(keep)
            def _():
                _st1(kb_v, [0, k], cy1)
                _st1(kb_v, [1, k], cx1)
                _st1(kb_v, [2, k], cy2)
                _st1(kb_v, [3, k], cx2)
                _st1(ks_v, [k], top)

            return (k + keep.astype(I32), valid.astype(I32))

        lax.while_loop(cond, body, (jnp.int32(0), jnp.int32(1)))

        pltpu.sync_copy(ks_v, oks_hbm.at[bb, pl.ds(cc * KS, KS)])
        pltpu.sync_copy(kb_v, okb_hbm.at[bb, cc])


def _sc_nms(scores, boxes):
    f = functools.partial(
        pl.kernel,
        mesh=_mesh(),
        compiler_params=pltpu.CompilerParams(needs_layout_passes=False),
        out_type=[
            jax.ShapeDtypeStruct((B, C * KS), F32),
            jax.ShapeDtypeStruct((B, C, 4, KS), F32),
        ],
        scratch_types=[
            pltpu.VMEM((NP,), F32),
            pltpu.VMEM((4, NP), F32),
            pltpu.VMEM((NV,), F32),
            pltpu.VMEM((L2N,), F32),
            pltpu.VMEM((16,), F32),
            pltpu.VMEM((KS,), F32),
            pltpu.VMEM((4, KS), F32),
            pltpu.SemaphoreType.DMA,
        ],
    )(_sc_nms_body)
    return f(scores, boxes)


def _sc_merge_body(ks_hbm, kb_hbm, ob_hbm, ol_hbm, os_hbm,
                   s4_v, b4_v, l1_v, ob_v, ol_v, os_v):
    wid = _wid()

    @pl.when(wid < B)
    def _():
        pltpu.sync_copy(ks_hbm.at[wid], s4_v)
        pltpu.sync_copy(kb_hbm.at[wid], b4_v)

        nv = C * KS // 16

        def build1(i, _):
            _st1(l1_v, [i], _smax(s4_v[pl.ds(i * 16, 16)]))
            return 0
        lax.fori_loop(0, nv, build1, 0)

        def zo(i, _):
            ob_v[pl.ds(i * 16, 16)] = jnp.zeros((16,), F32)
            return 0
        lax.fori_loop(0, 4 * M // 16, zo, 0)
        for i in range(6):
            ol_v[pl.ds(i * 16, 16)] = jnp.zeros((16,), F32)
            os_v[pl.ds(i * 16, 16)] = jnp.zeros((16,), F32)
        ol_v[pl.ds(104 - 16, 16)] = jnp.zeros((16,), F32)
        os_v[pl.ds(104 - 16, 16)] = jnp.zeros((16,), F32)

        def emit(i, _):
            va = l1_v[pl.ds(0, 16)]
            vb = l1_v[pl.ds(16, 16)]
            top = jnp.maximum(_smax(va), _smax(vb))
            fa = plsc.all_reduce_ffs(va == top)[0]
            fb = plsc.all_reduce_ffs(vb == top)[0]
            j = jnp.where(fa < 16, fa, 16 + fb)
            vsc = s4_v[pl.ds(j * 16, 16)]
            n = j * 16 + _arglane(vsc, top)
            valid = top > NEG / 2.0

            cls = lax.shift_right_logical(n, 7)
            slot = jnp.bitwise_and(n, KS - 1)
            y1 = _ld1(b4_v, [cls, 0, slot])
            x1 = _ld1(b4_v, [cls, 1, slot])
            y2 = _ld1(b4_v, [cls, 2, slot])
            x2 = _ld1(b4_v, [cls, 3, slot])

            @pl.when(valid)
            def _():
                _st1(s4_v, [n], jnp.float32(NEG))
                _st1(l1_v, [j], _smax(s4_v[pl.ds(j * 16, 16)]))
                _st1(ob_v, [4 * i], y1)
                _st1(ob_v, [4 * i + 1], x1)
                _st1(ob_v, [4 * i + 2], y2)
                _st1(ob_v, [4 * i + 3], x2)
                _st1(ol_v, [i], cls.astype(F32))
                _st1(os_v, [i], top)
            return 0

        lax.fori_loop(0, M, emit, 0)

        pltpu.sync_copy(ob_v, ob_hbm.at[wid])
        pltpu.sync_copy(ol_v, ol_hbm.at[wid])
        pltpu.sync_copy(os_v, os_hbm.at[wid])


def _sc_merge(kscores, kboxes):
    f = functools.partial(
        pl.kernel,
        mesh=_mesh(),
        compiler_params=pltpu.CompilerParams(needs_layout_passes=False),
        out_type=[
            jax.ShapeDtypeStruct((B, 4 * M), F32),
            jax.ShapeDtypeStruct((B, 104), F32),
            jax.ShapeDtypeStruct((B, 104), F32),
        ],
        scratch_types=[
            pltpu.VMEM((C * KS,), F32),
            pltpu.VMEM((C, 4, KS), F32),
            pltpu.VMEM((C * KS // 16,), F32),
            pltpu.VMEM((4 * M,), F32),
            pltpu.VMEM((104,), F32),
            pltpu.VMEM((104,), F32),
        ],
    )(_sc_merge_body)
    return f(kscores, kboxes)


def kernel(roi_bboxes, pred_deltas, pred_label_probs):
    pad = NP - N
    roi_t = jnp.pad(jnp.transpose(roi_bboxes, (0, 2, 1)), ((0, 0), (0, 0), (0, pad)))
    d = pred_deltas.reshape(B, N, C, 4)
    d_t = jnp.pad(jnp.transpose(d, (0, 2, 3, 1)), ((0, 0), (0, 0), (0, 0), (0, pad)))
    p_t = jnp.pad(jnp.transpose(pred_label_probs, (0, 2, 1)), ((0, 0), (0, 0), (0, pad)))

    scores, boxes = _tc_decode(roi_t, d_t, p_t)
    scores = scores.reshape(B * C, NP)
    boxes = boxes.reshape(B * C, 4, NP)
    ks, kb = _sc_nms(scores, boxes)
    ob, ol, os_ = _sc_merge(ks, kb)

    final_boxes = ob.reshape(B, M, 4)
    final_labels = ol[:, :M]
    final_scores = os_[:, :M]
    return (final_boxes, final_labels, final_scores)


# speedup vs baseline: 114.5568x; 1.0552x over previous
import functools

import jax
import jax.numpy as jnp
from jax import lax
from jax.experimental import pallas as pl
from jax.experimental.pallas import tpu as pltpu
from jax.experimental.pallas import tpu_sc as plsc

NEG = -1e9
B = 4
C = 4
N = 20000
NP = 20480
NV = NP // 16
L2N = NV // 16
M = 100
KS = 128
SCORE_T = 0.05
IOU_T = 0.5
F32 = jnp.float32
I32 = jnp.int32


def _tc_decode_body(roi_ref, d_ref, p_ref, sc_ref, bx_ref):
    ay1 = roi_ref[0, 0:1, :]
    ax1 = roi_ref[0, 1:2, :]
    ay2 = roi_ref[0, 2:3, :]
    ax2 = roi_ref[0, 3:4, :]
    aw = ax2 - ax1
    ah = ay2 - ay1
    acx = ax1 + 0.5 * aw
    acy = ay1 + 0.5 * ah
    p0 = p_ref[0, 0:1, :]
    p1 = p_ref[0, 1:2, :]
    p2 = p_ref[0, 2:3, :]
    p3 = p_ref[0, 3:4, :]
    bg = p0 >= jnp.maximum(p1, jnp.maximum(p2, p3))
    for c in range(C):
        dy = d_ref[0, c, 0:1, :] * 0.1
        dx = d_ref[0, c, 1:2, :] * 0.1
        dh = d_ref[0, c, 2:3, :] * 0.2
        dw = d_ref[0, c, 3:4, :] * 0.2
        bw = jnp.exp(dw) * aw
        bh = jnp.exp(dh) * ah
        bcx = dx * aw + acx
        bcy = dy * ah + acy
        y1 = bcy - 0.5 * bh
        x1 = bcx - 0.5 * bw
        y2 = bh + y1
        x2 = bw + x1
        bx_ref[0, c, 0:1, :] = jnp.clip(y1, 0.0, 1.0)
        bx_ref[0, c, 1:2, :] = jnp.clip(x1, 0.0, 1.0)
        bx_ref[0, c, 2:3, :] = jnp.clip(y2, 0.0, 1.0)
        bx_ref[0, c, 3:4, :] = jnp.clip(x2, 0.0, 1.0)
        pc = (p0, p1, p2, p3)[c]
        s = jnp.where(bg, 0.0, pc)
        sc_ref[0, c:c + 1, :] = jnp.where(s > SCORE_T, s, NEG)


def _tc_decode(roi_t, d_t, p_t):
    return pl.pallas_call(
        _tc_decode_body,
        grid=(B,),
        in_specs=[
            pl.BlockSpec((1, 4, NP), lambda g: (g, 0, 0)),
            pl.BlockSpec((1, C, 4, NP), lambda g: (g, 0, 0, 0)),
            pl.BlockSpec((1, 4, NP), lambda g: (g, 0, 0)),
        ],
        out_specs=[
            pl.BlockSpec((1, C, NP), lambda g: (g, 0, 0)),
            pl.BlockSpec((1, C, 4, NP), lambda g: (g, 0, 0, 0)),
        ],
        out_shape=[
            jax.ShapeDtypeStruct((B, C, NP), F32),
            jax.ShapeDtypeStruct((B, C, 4, NP), F32),
        ],
    )(roi_t, d_t, p_t)


def _wid():
    return lax.axis_index("s") * 2 + lax.axis_index("c")


def _smax(v):
    return lax.reduce_max(v, (0,))


def _arglane(v, top):
    return plsc.all_reduce_ffs(v == top)[0]


def _bc(x):
    return jnp.broadcast_to(x, (16,))


def _st1(ref, idxs, val):
    mask = lax.iota(I32, 16) < 1
    plsc.store_scatter(ref, [_bc(jnp.asarray(i, I32)) for i in idxs],
                       _bc(val), mask=mask)


def _ld1(ref, idxs):
    return plsc.load_gather(ref, [_bc(jnp.asarray(i, I32)) for i in idxs])


def _mesh():
    return plsc.VectorSubcoreMesh(core_axis_name="c", subcore_axis_name="s")


def _sc_nms_body(sc_hbm, bx_hbm, ob_hbm, ol_hbm, os_hbm,
                 s_v, b_v, l1_v, l2_v, l3_v, ks_v, kb_v,
                 shb_sh, shs_sh, mb_v, ms_v, l1m_v, ob_v, ol_v, os_v, sem):
    core = lax.axis_index("c")
    sub = lax.axis_index("s")
    bslot = lax.shift_right_logical(sub, 2)

    @pl.when(sub < 8)
    def _():
        cc = jnp.bitwise_and(sub, 3)
        wid = (core * 2 + bslot) * C + cc
        box_cp = pltpu.async_copy(bx_hbm.at[wid], b_v, sem)
        pltpu.sync_copy(sc_hbm.at[wid], s_v)

        for i in range(KS // 16):
            kb_v[0, pl.ds(i * 16, 16)] = jnp.full((16,), -4.0, F32)
            kb_v[1, pl.ds(i * 16, 16)] = jnp.full((16,), -4.0, F32)
            kb_v[2, pl.ds(i * 16, 16)] = jnp.full((16,), -4.0, F32)
            kb_v[3, pl.ds(i * 16, 16)] = jnp.full((16,), -4.0, F32)
            ks_v[pl.ds(i * 16, 16)] = jnp.full((16,), NEG, F32)

        def build1(i, _):
            for u in range(8):
                g = i * 8 + u
                _st1(l1_v, [g], _smax(s_v[pl.ds(g * 16, 16)]))
            return 0
        lax.fori_loop(0, NV // 8, build1, 0)

        def build2(i, _):
            for u in range(8):
                g = i * 8 + u
                _st1(l2_v, [g], _smax(l1_v[pl.ds(g * 16, 16)]))
            return 0
        lax.fori_loop(0, L2N // 8, build2, 0)

        l3_v[pl.ds(0, 16)] = jnp.full((16,), NEG, F32)
        for g in range(L2N // 16):
            _st1(l3_v, [g], _smax(l2_v[pl.ds(g * 16, 16)]))

        box_cp.wait()

        def cond(carry):
            k, cont = carry
            return jnp.logical_and(k < M, cont > 0)

        def body(carry):
            k, _ = carry
            ii = lax.iota(I32, 16)
            v3 = l3_v[pl.ds(0, 16)]
            top = _smax(v3)
            valid = top > NEG / 2.0
            l3 = _arglane(v3, top)
            v2 = l2_v[pl.ds(l3 * 16, 16)]
            i2 = _arglane(v2, top)
            j1 = l3 * 16 + i2
            v1 = l1_v[pl.ds(j1 * 16, 16)]
            i1 = _arglane(v1, top)
            blk = j1 * 16 + i1
            vs = s_v[pl.ds(blk * 16, 16)]
            i0 = _arglane(vs, top)
            n = blk * 16 + i0

            cy1 = _ld1(b_v, [0, n])
            cx1 = _ld1(b_v, [1, n])
            cy2 = _ld1(b_v, [2, n])
            cx2 = _ld1(b_v, [3, n])
            a1 = jnp.maximum(cy2 - cy1, 0.0) * jnp.maximum(cx2 - cx1, 0.0)

            def iou_vreg(vv, acc):
                ky1 = kb_v[0, pl.ds(vv * 16, 16)]
                kx1 = kb_v[1, pl.ds(vv * 16, 16)]
                ky2 = kb_v[2, pl.ds(vv * 16, 16)]
                kx2 = kb_v[3, pl.ds(vv * 16, 16)]
                yy1 = jnp.maximum(cy1, ky1)
                xx1 = jnp.maximum(cx1, kx1)
                yy2 = jnp.minimum(cy2, ky2)
                xx2 = jnp.minimum(cx2, kx2)
                inter = jnp.maximum(yy2 - yy1, 0.0) * jnp.maximum(xx2 - xx1, 0.0)
                a2 = jnp.maximum(ky2 - ky1, 0.0) * jnp.maximum(kx2 - kx1, 0.0)
                union = a1 + a2 - inter
                iou = jnp.where(union > 0.0, inter / union, 0.0)
                return jnp.maximum(acc, iou)

            kv = lax.shift_right_logical(k + 15, 4)
            acc = lax.fori_loop(0, kv, iou_vreg, jnp.zeros((16,), F32))
            sup = _smax(acc) > IOU_T
            keep = jnp.logical_and(valid, jnp.logical_not(sup))

            @pl.when(valid)
            def _():
                vs2 = jnp.where(ii == i0, NEG, vs)
                s_v[pl.ds(blk * 16, 16)] = vs2
                m1 = _smax(vs2)
                v1n = jnp.where(ii == i1, m1, v1)
                l1_v[pl.ds(j1 * 16, 16)] = v1n
                m2 = _smax(v1n)
                v2n = jnp.where(ii == i2, m2, v2)
                l2_v[pl.ds(l3 * 16, 16)] = v2n
                m3 = _smax(v2n)
                l3_v[pl.ds(0, 16)] = jnp.where(ii == l3, m3, v3)

            @pl.when(keep)
            def _():
                _st1(kb_v, [0, k], cy1)
                _st1(kb_v, [1, k], cx1)
                _st1(kb_v, [2, k], cy2)
                _st1(kb_v, [3, k], cx2)
                _st1(ks_v, [k], top)

            return (k + keep.astype(I32), valid.astype(I32))

        lax.while_loop(cond, body, (jnp.int32(0), jnp.int32(1)))

        pltpu.sync_copy(kb_v, shb_sh.at[bslot, cc])
        pltpu.sync_copy(ks_v, shs_sh.at[bslot, cc])

    plsc.subcore_barrier()

    @pl.when(jnp.logical_and(sub >= 8, sub < 10))
    def _():
        ms = sub - 8
        bout = core * 2 + ms
        pltpu.sync_copy(shb_sh.at[ms], mb_v)
        for c0 in range(C):
            pltpu.sync_copy(shs_sh.at[ms, c0], ms_v.at[pl.ds(c0 * KS, KS)])

        nv = C * KS // 16

        def build1(i, _):
            _st1(l1m_v, [i], _smax(ms_v[pl.ds(i * 16, 16)]))
            return 0
        lax.fori_loop(0, nv, build1, 0)

        def zo(i, _):
            ob_v[pl.ds(i * 16, 16)] = jnp.zeros((16,), F32)
            return 0
        lax.fori_loop(0, 4 * M // 16, zo, 0)
        for i in range(6):
            ol_v[pl.ds(i * 16, 16)] = jnp.zeros((16,), F32)
            os_v[pl.ds(i * 16, 16)] = jnp.zeros((16,), F32)
        ol_v[pl.ds(104 - 16, 16)] = jnp.zeros((16,), F32)
        os_v[pl.ds(104 - 16, 16)] = jnp.zeros((16,), F32)

        def emit(i, _):
            va = l1m_v[pl.ds(0, 16)]
            vb = l1m_v[pl.ds(16, 16)]
            top = jnp.maximum(_smax(va), _smax(vb))
            fa = _arglane(va, top)
            fb = _arglane(vb, top)
            j = jnp.where(fa < 16, fa, 16 + fb)
            vsc = ms_v[pl.ds(j * 16, 16)]
            n = j * 16 + _arglane(vsc, top)
            valid = top > NEG / 2.0

            cls = lax.shift_right_logical(n, 7)
            slot = jnp.bitwise_and(n, KS - 1)
            y1 = _ld1(mb_v, [cls, 0, slot])
            x1 = _ld1(mb_v, [cls, 1, slot])
            y2 = _ld1(mb_v, [cls, 2, slot])
            x2 = _ld1(mb_v, [cls, 3, slot])

            @pl.when(valid)
            def _():
                _st1(ms_v, [n], jnp.float32(NEG))
                _st1(l1m_v, [j], _smax(ms_v[pl.ds(j * 16, 16)]))
                _st1(ob_v, [4 * i], y1)
                _st1(ob_v, [4 * i + 1], x1)
                _st1(ob_v, [4 * i + 2], y2)
                _st1(ob_v, [4 * i + 3], x2)
                _st1(ol_v, [i], cls.astype(F32))
                _st1(os_v, [i], top)
            return 0

        lax.fori_loop(0, M, emit, 0)

        pltpu.sync_copy(ob_v, ob_hbm.at[bout])
        pltpu.sync_copy(ol_v, ol_hbm.at[bout])
        pltpu.sync_copy(os_v, os_hbm.at[bout])


def _sc_nms(scores, boxes):
    f = functools.partial(
        pl.kernel,
        mesh=_mesh(),
        compiler_params=pltpu.CompilerParams(needs_layout_passes=False),
        out_type=[
            jax.ShapeDtypeStruct((B, 4 * M), F32),
            jax.ShapeDtypeStruct((B, 104), F32),
            jax.ShapeDtypeStruct((B, 104), F32),
        ],
        scratch_types=[
            pltpu.VMEM((NP,), F32),
            pltpu.VMEM((4, NP), F32),
            pltpu.VMEM((NV,), F32),
            pltpu.VMEM((L2N,), F32),
            pltpu.VMEM((16,), F32),
            pltpu.VMEM((KS,), F32),
            pltpu.VMEM((4, KS), F32),
            pltpu.VMEM_SHARED((2, C, 4, KS), F32),
            pltpu.VMEM_SHARED((2, C, KS), F32),
            pltpu.VMEM((C, 4, KS), F32),
            pltpu.VMEM((C * KS,), F32),
            pltpu.VMEM((C * KS // 16,), F32),
            pltpu.VMEM((4 * M,), F32),
            pltpu.VMEM((104,), F32),
            pltpu.VMEM((104,), F32),
            pltpu.SemaphoreType.DMA,
        ],
    )(_sc_nms_body)
    return f(scores, boxes)


def kernel(roi_bboxes, pred_deltas, pred_label_probs):
    pad = NP - N
    roi_t = jnp.pad(jnp.transpose(roi_bboxes, (0, 2, 1)), ((0, 0), (0, 0), (0, pad)))
    d = pred_deltas.reshape(B, N, C, 4)
    d_t = jnp.pad(jnp.transpose(d, (0, 2, 3, 1)), ((0, 0), (0, 0), (0, 0), (0, pad)))
    p_t = jnp.pad(jnp.transpose(pred_label_probs, (0, 2, 1)), ((0, 0), (0, 0), (0, pad)))

    scores, boxes = _tc_decode(roi_t, d_t, p_t)
    scores = scores.reshape(B * C, NP)
    boxes = boxes.reshape(B * C, 4, NP)
    ob, ol, os_ = _sc_nms(scores, boxes)

    final_boxes = ob.reshape(B, M, 4)
    final_labels = ol[:, :M]
    final_scores = os_[:, :M]
    return (final_boxes, final_labels, final_scores)


# speedup vs baseline: 115.1118x; 1.0048x over previous
import functools

import jax
import jax.numpy as jnp
from jax import lax
from jax.experimental import pallas as pl
from jax.experimental.pallas import tpu as pltpu
from jax.experimental.pallas import tpu_sc as plsc

NEG = -1e9
B = 4
C = 4
N = 20000
NP = 20480
NV = NP // 16
L2N = NV // 16
M = 100
KS = 128
SCORE_T = 0.05
IOU_T = 0.5
F32 = jnp.float32
I32 = jnp.int32


def _tc_decode_body(roi_ref, d_ref, p_ref, sc_ref, bx_ref):
    ay1 = roi_ref[0, 0:1, :]
    ax1 = roi_ref[0, 1:2, :]
    ay2 = roi_ref[0, 2:3, :]
    ax2 = roi_ref[0, 3:4, :]
    aw = ax2 - ax1
    ah = ay2 - ay1
    acx = ax1 + 0.5 * aw
    acy = ay1 + 0.5 * ah
    p0 = p_ref[0, 0:1, :]
    p1 = p_ref[0, 1:2, :]
    p2 = p_ref[0, 2:3, :]
    p3 = p_ref[0, 3:4, :]
    bg = p0 >= jnp.maximum(p1, jnp.maximum(p2, p3))
    for c in range(C):
        dy = d_ref[0, c, 0:1, :] * 0.1
        dx = d_ref[0, c, 1:2, :] * 0.1
        dh = d_ref[0, c, 2:3, :] * 0.2
        dw = d_ref[0, c, 3:4, :] * 0.2
        bw = jnp.exp(dw) * aw
        bh = jnp.exp(dh) * ah
        bcx = dx * aw + acx
        bcy = dy * ah + acy
        y1 = bcy - 0.5 * bh
        x1 = bcx - 0.5 * bw
        y2 = bh + y1
        x2 = bw + x1
        bx_ref[0, c, 0:1, :] = jnp.clip(y1, 0.0, 1.0)
        bx_ref[0, c, 1:2, :] = jnp.clip(x1, 0.0, 1.0)
        bx_ref[0, c, 2:3, :] = jnp.clip(y2, 0.0, 1.0)
        bx_ref[0, c, 3:4, :] = jnp.clip(x2, 0.0, 1.0)
        pc = (p0, p1, p2, p3)[c]
        s = jnp.where(bg, 0.0, pc)
        sc_ref[0, c:c + 1, :] = jnp.where(s > SCORE_T, s, NEG)


def _tc_decode(roi_t, d_t, p_t):
    return pl.pallas_call(
        _tc_decode_body,
        grid=(B,),
        in_specs=[
            pl.BlockSpec((1, 4, NP), lambda g: (g, 0, 0)),
            pl.BlockSpec((1, C, 4, NP), lambda g: (g, 0, 0, 0)),
            pl.BlockSpec((1, 4, NP), lambda g: (g, 0, 0)),
        ],
        out_specs=[
            pl.BlockSpec((1, C, NP), lambda g: (g, 0, 0)),
            pl.BlockSpec((1, C, 4, NP), lambda g: (g, 0, 0, 0)),
        ],
        out_shape=[
            jax.ShapeDtypeStruct((B, C, NP), F32),
            jax.ShapeDtypeStruct((B, C, 4, NP), F32),
        ],
    )(roi_t, d_t, p_t)


def _wid():
    return lax.axis_index("s") * 2 + lax.axis_index("c")


def _smax(v):
    return lax.reduce_max(v, (0,))


def _arglane(v, top):
    return plsc.all_reduce_ffs(v == top)[0]


def _bc(x):
    return jnp.broadcast_to(x, (16,))


def _st1(ref, idxs, val):
    mask = lax.iota(I32, 16) < 1
    plsc.store_scatter(ref, [_bc(jnp.asarray(i, I32)) for i in idxs],
                       _bc(val), mask=mask)


def _ld1(ref, idxs):
    return plsc.load_gather(ref, [_bc(jnp.asarray(i, I32)) for i in idxs])


def _mesh():
    return plsc.VectorSubcoreMesh(core_axis_name="c", subcore_axis_name="s")


def _sc_nms_body(sc_hbm, bx_hbm, ob_hbm, ol_hbm, os_hbm,
                 s_v, b_v, l1_v, l2_v, l3_v, ks_v, kb_v,
                 shb_sh, shs_sh, mb_v, ms_v, l1m_v, ob_v, ol_v, os_v,
                 sem, sem2):
    core = lax.axis_index("c")
    sub = lax.axis_index("s")
    bslot = lax.shift_right_logical(sub, 2)

    @pl.when(sub < 8)
    def _():
        cc = jnp.bitwise_and(sub, 3)
        wid = (core * 2 + bslot) * C + cc
        box_cp = pltpu.async_copy(bx_hbm.at[wid], b_v, sem)
        sc_cp = pltpu.async_copy(sc_hbm.at[wid], s_v, sem2)

        for i in range(KS // 16):
            kb_v[0, pl.ds(i * 16, 16)] = jnp.full((16,), -4.0, F32)
            kb_v[1, pl.ds(i * 16, 16)] = jnp.full((16,), -4.0, F32)
            kb_v[2, pl.ds(i * 16, 16)] = jnp.full((16,), -4.0, F32)
            kb_v[3, pl.ds(i * 16, 16)] = jnp.full((16,), -4.0, F32)
            ks_v[pl.ds(i * 16, 16)] = jnp.full((16,), NEG, F32)

        sc_cp.wait()

        def build1(i, _):
            for u in range(8):
                g = i * 8 + u
                _st1(l1_v, [g], _smax(s_v[pl.ds(g * 16, 16)]))
            return 0
        lax.fori_loop(0, NV // 8, build1, 0)

        def build2(i, _):
            for u in range(8):
                g = i * 8 + u
                _st1(l2_v, [g], _smax(l1_v[pl.ds(g * 16, 16)]))
            return 0
        lax.fori_loop(0, L2N // 8, build2, 0)

        l3_v[pl.ds(0, 16)] = jnp.full((16,), NEG, F32)
        for g in range(L2N // 16):
            _st1(l3_v, [g], _smax(l2_v[pl.ds(g * 16, 16)]))

        box_cp.wait()

        def cond(carry):
            k, cont = carry
            return jnp.logical_and(k < M, cont > 0)

        def body(carry):
            k, _ = carry
            ii = lax.iota(I32, 16)
            v3 = l3_v[pl.ds(0, 16)]
            top = _smax(v3)
            valid = top > NEG / 2.0
            l3 = _arglane(v3, top)
            v2 = l2_v[pl.ds(l3 * 16, 16)]
            i2 = _arglane(v2, top)
            j1 = l3 * 16 + i2
            v1 = l1_v[pl.ds(j1 * 16, 16)]
            i1 = _arglane(v1, top)
            blk = j1 * 16 + i1
            vs = s_v[pl.ds(blk * 16, 16)]
            i0 = _arglane(vs, top)
            n = blk * 16 + i0

            cy1 = _ld1(b_v, [0, n])
            cx1 = _ld1(b_v, [1, n])
            cy2 = _ld1(b_v, [2, n])
            cx2 = _ld1(b_v, [3, n])
            a1 = jnp.maximum(cy2 - cy1, 0.0) * jnp.maximum(cx2 - cx1, 0.0)

            acc = jnp.zeros((16,), F32)
            for vv in range(KS // 16):
                ky1 = kb_v[0, pl.ds(vv * 16, 16)]
                kx1 = kb_v[1, pl.ds(vv * 16, 16)]
                ky2 = kb_v[2, pl.ds(vv * 16, 16)]
                kx2 = kb_v[3, pl.ds(vv * 16, 16)]
                yy1 = jnp.maximum(cy1, ky1)
                xx1 = jnp.maximum(cx1, kx1)
                yy2 = jnp.minimum(cy2, ky2)
                xx2 = jnp.minimum(cx2, kx2)
                inter = jnp.maximum(yy2 - yy1, 0.0) * jnp.maximum(xx2 - xx1, 0.0)
                a2 = jnp.maximum(ky2 - ky1, 0.0) * jnp.maximum(kx2 - kx1, 0.0)
                union = a1 + a2 - inter
                iou = jnp.where(union > 0.0, inter / union, 0.0)
                acc = jnp.maximum(acc, iou)
            sup = _smax(acc) > IOU_T
            keep = jnp.logical_and(valid, jnp.logical_not(sup))

            vs2 = jnp.where(ii == i0, NEG, vs)
            s_v[pl.ds(blk * 16, 16)] = vs2
            m1 = _smax(vs2)
            v1n = jnp.where(ii == i1, m1, v1)
            l1_v[pl.ds(j1 * 16, 16)] = v1n
            m2 = _smax(v1n)
            v2n = jnp.where(ii == i2, m2, v2)
            l2_v[pl.ds(l3 * 16, 16)] = v2n
            m3 = _smax(v2n)
            l3_v[pl.ds(0, 16)] = jnp.where(ii == l3, m3, v3)

            @pl.when(keep)
            def _():
                _st1(kb_v, [0, k], cy1)
                _st1(kb_v, [1, k], cx1)
                _st1(kb_v, [2, k], cy2)
                _st1(kb_v, [3, k], cx2)
                _st1(ks_v, [k], top)

            return (k + keep.astype(I32), valid.astype(I32))

        lax.while_loop(cond, body, (jnp.int32(0), jnp.int32(1)))

        pltpu.sync_copy(kb_v, shb_sh.at[bslot, cc])
        pltpu.sync_copy(ks_v, shs_sh.at[bslot, cc])

    plsc.subcore_barrier()

    @pl.when(jnp.logical_and(sub >= 8, sub < 10))
    def _():
        ms = sub - 8
        bout = core * 2 + ms
        pltpu.sync_copy(shb_sh.at[ms], mb_v)
        for c0 in range(C):
            pltpu.sync_copy(shs_sh.at[ms, c0], ms_v.at[pl.ds(c0 * KS, KS)])

        nv = C * KS // 16

        def build1(i, _):
            _st1(l1m_v, [i], _smax(ms_v[pl.ds(i * 16, 16)]))
            return 0
        lax.fori_loop(0, nv, build1, 0)

        def zo(i, _):
            ob_v[pl.ds(i * 16, 16)] = jnp.zeros((16,), F32)
            return 0
        lax.fori_loop(0, 4 * M // 16, zo, 0)
        for i in range(6):
            ol_v[pl.ds(i * 16, 16)] = jnp.zeros((16,), F32)
            os_v[pl.ds(i * 16, 16)] = jnp.zeros((16,), F32)
        ol_v[pl.ds(104 - 16, 16)] = jnp.zeros((16,), F32)
        os_v[pl.ds(104 - 16, 16)] = jnp.zeros((16,), F32)

        def emit(i, _):
            va = l1m_v[pl.ds(0, 16)]
            vb = l1m_v[pl.ds(16, 16)]
            top = jnp.maximum(_smax(va), _smax(vb))
            fa = _arglane(va, top)
            fb = _arglane(vb, top)
            j = jnp.where(fa < 16, fa, 16 + fb)
            vsc = ms_v[pl.ds(j * 16, 16)]
            n = j * 16 + _arglane(vsc, top)
            valid = top > NEG / 2.0

            cls = lax.shift_right_logical(n, 7)
            slot = jnp.bitwise_and(n, KS - 1)
            y1 = _ld1(mb_v, [cls, 0, slot])
            x1 = _ld1(mb_v, [cls, 1, slot])
            y2 = _ld1(mb_v, [cls, 2, slot])
            x2 = _ld1(mb_v, [cls, 3, slot])

            @pl.when(valid)
            def _():
                _st1(ms_v, [n], jnp.float32(NEG))
                _st1(l1m_v, [j], _smax(ms_v[pl.ds(j * 16, 16)]))
                _st1(ob_v, [4 * i], y1)
                _st1(ob_v, [4 * i + 1], x1)
                _st1(ob_v, [4 * i + 2], y2)
                _st1(ob_v, [4 * i + 3], x2)
                _st1(ol_v, [i], cls.astype(F32))
                _st1(os_v, [i], top)
            return 0

        lax.fori_loop(0, M, emit, 0)

        pltpu.sync_copy(ob_v, ob_hbm.at[bout])
        pltpu.sync_copy(ol_v, ol_hbm.at[bout])
        pltpu.sync_copy(os_v, os_hbm.at[bout])


def _sc_nms(scores, boxes):
    f = functools.partial(
        pl.kernel,
        mesh=_mesh(),
        compiler_params=pltpu.CompilerParams(needs_layout_passes=False),
        out_type=[
            jax.ShapeDtypeStruct((B, 4 * M), F32),
            jax.ShapeDtypeStruct((B, 104), F32),
            jax.ShapeDtypeStruct((B, 104), F32),
        ],
        scratch_types=[
            pltpu.VMEM((NP,), F32),
            pltpu.VMEM((4, NP), F32),
            pltpu.VMEM((NV,), F32),
            pltpu.VMEM((L2N,), F32),
            pltpu.VMEM((16,), F32),
            pltpu.VMEM((KS,), F32),
            pltpu.VMEM((4, KS), F32),
            pltpu.VMEM_SHARED((2, C, 4, KS), F32),
            pltpu.VMEM_SHARED((2, C, KS), F32),
            pltpu.VMEM((C, 4, KS), F32),
            pltpu.VMEM((C * KS,), F32),
            pltpu.VMEM((C * KS // 16,), F32),
            pltpu.VMEM((4 * M,), F32),
            pltpu.VMEM((104,), F32),
            pltpu.VMEM((104,), F32),
            pltpu.SemaphoreType.DMA,
            pltpu.SemaphoreType.DMA,
        ],
    )(_sc_nms_body)
    return f(scores, boxes)


def kernel(roi_bboxes, pred_deltas, pred_label_probs):
    pad = NP - N
    roi_t = jnp.pad(jnp.transpose(roi_bboxes, (0, 2, 1)), ((0, 0), (0, 0), (0, pad)))
    d = pred_deltas.reshape(B, N, C, 4)
    d_t = jnp.pad(jnp.transpose(d, (0, 2, 3, 1)), ((0, 0), (0, 0), (0, 0), (0, pad)))
    p_t = jnp.pad(jnp.transpose(pred_label_probs, (0, 2, 1)), ((0, 0), (0, 0), (0, pad)))

    scores, boxes = _tc_decode(roi_t, d_t, p_t)
    scores = scores.reshape(B * C, NP)
    boxes = boxes.reshape(B * C, 4, NP)
    ob, ol, os_ = _sc_nms(scores, boxes)

    final_boxes = ob.reshape(B, M, 4)
    final_labels = ol[:, :M]
    final_scores = os_[:, :M]
    return (final_boxes, final_labels, final_scores)
